# baseline (speedup 1.0000x reference)
import functools

import jax
import jax.numpy as jnp
from jax import lax
from jax.experimental import pallas as pl
from jax.experimental.pallas import tpu as pltpu
from jax.experimental.pallas import tpu_sc as plsc

RADIUS = 0.2
N = 1024
CT = 1024
OUT_CH = 128
HIGH = jax.lax.Precision.HIGHEST


def _tc_kernel(xyzT_ref, xyzS_ref, W1_ref, W2_ref, W3_ref,
               b1_ref, b2_ref, b3_ref, idx_ref, A_ref, C_ref):
    bt = pl.program_id(0)
    ct = pl.program_id(1)
    judge = jnp.float32(RADIUS * RADIUS)

    W21 = jnp.dot(W2_ref[...], W1_ref[...], preferred_element_type=jnp.float32, precision=HIGH)
    W321 = jnp.dot(W3_ref[...], W21, preferred_element_type=jnp.float32, precision=HIGH)
    be = jax.lax.dot_general(b1_ref[...], W2_ref[...], (((1,), (1,)), ((), ())), precision=HIGH) + b2_ref[...]
    beff = jax.lax.dot_general(be, W3_ref[...], (((1,), (1,)), ((), ())), precision=HIGH) + b3_ref[...]

    xyz_all = xyzS_ref[0]
    P = jax.lax.dot_general(xyz_all, W321, (((1,), (1,)), ((), ())),
                            preferred_element_type=jnp.float32, precision=HIGH)
    A_ref[0] = P

    xs = xyzT_ref[0, 0:1, :]
    ys = xyzT_ref[0, 1:2, :]
    zs = xyzT_ref[0, 2:3, :]
    c3 = xyzS_ref[0, pl.ds(ct * CT, CT), :]
    xc = c3[:, 0:1]
    yc = c3[:, 1:2]
    zc = c3[:, 2:3]

    dx = xs - xc
    dy = ys - yc
    dz = zs - zc
    dist = dx * dx + dy * dy + dz * dz

    i0 = jnp.int32(0)
    neg_eps = jnp.float32(-(2.0 ** -25))
    s = (jnp.where(dx >= neg_eps, jnp.int32(4), i0)
         + jnp.where(dy >= neg_eps, jnp.int32(2), i0)
         + jnp.where(dz >= neg_eps, jnp.int32(1), i0))
    valid = (dist > jnp.float32(1e-10)) & (dist < judge)

    lane = jax.lax.broadcasted_iota(jnp.int32, (CT, N), 1)
    row = jax.lax.broadcasted_iota(jnp.int32, (CT, N), 0)
    eye = lane == (row + ct * CT)
    base = jnp.where(eye, judge, jnp.float32(1e10))
    dv = jnp.where(valid, dist, base)

    Pc = jax.lax.dot_general(c3, W321, (((1,), (1,)), ((), ())),
                             preferred_element_type=jnp.float32, precision=HIGH)
    C_ref[0] = beff - Pc

    gbase = bt * jnp.int32(N)
    for i in range(8):
        d_i = jnp.where(s == i, dv, base)
        mn = jnp.min(d_i, axis=1, keepdims=True)
        idx = jnp.min(jnp.where(d_i == mn, lane, N), axis=1, keepdims=True)
        idx_ref[0, :, i:i + 1] = idx + gbase


def _tc_stage(xyzT, xyzS, W1, W2, W3, b1r, b2r, b3r):
    BT = xyzS.shape[0]
    return pl.pallas_call(
        _tc_kernel,
        grid=(BT, N // CT),
        in_specs=[
            pl.BlockSpec((1, 3, N), lambda b, ct: (b, 0, 0)),
            pl.BlockSpec((1, N, 3), lambda b, ct: (b, 0, 0)),
            pl.BlockSpec((OUT_CH, 3), lambda b, ct: (0, 0)),
            pl.BlockSpec((OUT_CH, OUT_CH), lambda b, ct: (0, 0)),
            pl.BlockSpec((OUT_CH, OUT_CH), lambda b, ct: (0, 0)),
            pl.BlockSpec((1, OUT_CH), lambda b, ct: (0, 0)),
            pl.BlockSpec((1, OUT_CH), lambda b, ct: (0, 0)),
            pl.BlockSpec((1, OUT_CH), lambda b, ct: (0, 0)),
        ],
        out_specs=[
            pl.BlockSpec((1, CT, 8), lambda b, ct: (b, ct, 0)),
            pl.BlockSpec((1, N, OUT_CH), lambda b, ct: (b, 0, 0)),
            pl.BlockSpec((1, CT, OUT_CH), lambda b, ct: (b, ct, 0)),
        ],
        out_shape=[
            jax.ShapeDtypeStruct((BT, N, 8), jnp.int32),
            jax.ShapeDtypeStruct((BT, N, OUT_CH), jnp.float32),
            jax.ShapeDtypeStruct((BT, N, OUT_CH), jnp.float32),
        ],
        compiler_params=pltpu.CompilerParams(
            dimension_semantics=("parallel", "parallel")),
    )(xyzT, xyzS, W1, W2, W3, b1r, b2r, b3r)


NPTS = 8192
PCHUNK = 16
NW = 32


def _sc_stage(A, C, idx_flat):
    mesh = plsc.VectorSubcoreMesh(core_axis_name="c", subcore_axis_name="s")
    pts_per_w = NPTS // NW
    nchunks = pts_per_w // PCHUNK

    @functools.partial(
        pl.kernel, mesh=mesh,
        out_type=jax.ShapeDtypeStruct((NPTS, 14 * OUT_CH), jnp.float32),
        scratch_types=[
            pltpu.VMEM((PCHUNK * 8,), jnp.int32),
            pltpu.VMEM((PCHUNK * 8, OUT_CH), jnp.float32),
            pltpu.VMEM((PCHUNK, OUT_CH), jnp.float32),
            pltpu.VMEM((PCHUNK, 14 * OUT_CH), jnp.float32),
            pltpu.SemaphoreType.DMA,
        ],
    )
    def k(A_hbm, C_hbm, idx_hbm, out_hbm, idx_v, rows_v, c_v, out_v, sem):
        wid = lax.axis_index("s") * 2 + lax.axis_index("c")
        base = wid * pts_per_w

        def chunk_body(ci, _):
            p0 = base + ci * PCHUNK
            pltpu.sync_copy(idx_hbm.at[pl.ds(p0 * 8, PCHUNK * 8)], idx_v)
            pltpu.sync_copy(C_hbm.at[pl.ds(p0, PCHUNK)], c_v)
            pltpu.async_copy(A_hbm.at[idx_v], rows_v, sem).wait()

            def pt_body(p, _):
                for v in range(OUT_CH // 16):
                    sl = pl.ds(v * 16, 16)
                    cv = c_v[p, sl]
                    gk = [rows_v[p * 8 + kk, sl] + cv for kk in range(8)]
                    m01 = jnp.maximum(gk[0], gk[1])
                    m23 = jnp.maximum(gk[2], gk[3])
                    m45 = jnp.maximum(gk[4], gk[5])
                    m67 = jnp.maximum(gk[6], gk[7])
                    cols = [jnp.maximum(m01, m23), jnp.maximum(m45, m67),
                            m01, m23, m45, m67] + gk
                    for sslot, val in enumerate(cols):
                        out_v[p, pl.ds(sslot * OUT_CH + v * 16, 16)] = val
                return ()

            lax.fori_loop(0, PCHUNK, pt_body, ())
            pltpu.sync_copy(out_v, out_hbm.at[pl.ds(p0, PCHUNK)])
            return ()

        lax.fori_loop(0, nchunks, chunk_body, ())

    return k(A, C, idx_flat)


@jax.jit
def kernel(x, W1, b1, W2, b2, W3, b3):
    B, t, n, c = x.shape
    BT = B * t
    xyzS = x.reshape(BT, n, c)
    xyzT = jnp.transpose(xyzS, (0, 2, 1))
    b1r = b1.reshape(1, OUT_CH)
    b2r = b2.reshape(1, OUT_CH)
    b3r = b3.reshape(1, OUT_CH)

    idx, A, C = _tc_stage(xyzT, xyzS, W1, W2, W3, b1r, b2r, b3r)
    out = _sc_stage(A.reshape(BT * n, OUT_CH), C.reshape(BT * n, OUT_CH),
                    idx.reshape(BT * n * 8))
    return out.reshape(B, t, n, 14 * OUT_CH)


# speedup vs baseline: 1.3200x; 1.3200x over previous
import functools

import jax
import jax.numpy as jnp
from jax.experimental import pallas as pl
from jax.experimental.pallas import tpu as pltpu

RADIUS = 0.2
N = 1024
CT = 1024
OUT_CH = 128


def _tc_kernel(xyzT_ref, xyzS_ref, W1_ref, W2_ref, W3_ref,
               b1_ref, b2_ref, b3_ref, out_ref):
    ct = pl.program_id(1)
    judge = jnp.float32(RADIUS * RADIUS)

    W21 = jnp.dot(W2_ref[...], W1_ref[...], preferred_element_type=jnp.float32, precision=jax.lax.Precision.HIGHEST)
    W321 = jnp.dot(W3_ref[...], W21, preferred_element_type=jnp.float32, precision=jax.lax.Precision.HIGHEST)
    be = jax.lax.dot_general(b1_ref[...], W2_ref[...],
                             (((1,), (1,)), ((), ())), precision=jax.lax.Precision.HIGHEST) + b2_ref[...]
    beff = jax.lax.dot_general(be, W3_ref[...],
                               (((1,), (1,)), ((), ())), precision=jax.lax.Precision.HIGHEST) + b3_ref[...]

    xyz_all = xyzS_ref[0]
    P = jax.lax.dot_general(xyz_all, W321, (((1,), (1,)), ((), ())),
                            preferred_element_type=jnp.float32, precision=jax.lax.Precision.HIGHEST)
    P_hi = P.astype(jnp.bfloat16)
    P_lo = (P - P_hi.astype(jnp.float32)).astype(jnp.bfloat16)

    xs = xyzT_ref[0, 0:1, :]
    ys = xyzT_ref[0, 1:2, :]
    zs = xyzT_ref[0, 2:3, :]
    c3 = xyzS_ref[0, pl.ds(ct * CT, CT), :]
    xc = c3[:, 0:1]
    yc = c3[:, 1:2]
    zc = c3[:, 2:3]

    dx = xs - xc
    dy = ys - yc
    dz = zs - zc
    dist = dx * dx + dy * dy + dz * dz

    one = jnp.float32(1.0)
    i0 = jnp.int32(0)
    neg_eps = jnp.float32(-(2.0 ** -25))
    s = (jnp.where(dx >= neg_eps, jnp.int32(4), i0)
         + jnp.where(dy >= neg_eps, jnp.int32(2), i0)
         + jnp.where(dz >= neg_eps, jnp.int32(1), i0))
    valid = (dist > jnp.float32(1e-10)) & (dist < judge)

    lane = jax.lax.broadcasted_iota(jnp.int32, (CT, N), 1)
    row = jax.lax.broadcasted_iota(jnp.int32, (CT, N), 0)
    eye = lane == (row + ct * CT)
    base = jnp.where(eye, judge, jnp.float32(1e10))
    dv = jnp.where(valid, dist, base)

    Pc = jax.lax.dot_general(c3, W321, (((1,), (1,)), ((), ())),
                             preferred_element_type=jnp.float32, precision=jax.lax.Precision.HIGHEST)
    offs = beff - Pc

    g = []
    for i in range(8):
        d_i = jnp.where(s == i, dv, base)
        mn = jnp.min(d_i, axis=1, keepdims=True)
        idx = jnp.min(jnp.where(d_i == mn, lane, N), axis=1, keepdims=True)
        onehot = (lane == idx).astype(jnp.bfloat16)
        Gi = (jnp.dot(onehot, P_hi, preferred_element_type=jnp.float32)
              + jnp.dot(onehot, P_lo, preferred_element_type=jnp.float32))
        g.append(Gi + offs)

    m01 = jnp.maximum(g[0], g[1])
    m23 = jnp.maximum(g[2], g[3])
    m45 = jnp.maximum(g[4], g[5])
    m67 = jnp.maximum(g[6], g[7])
    q0 = jnp.maximum(m01, m23)
    q1 = jnp.maximum(m45, m67)

    C = OUT_CH
    cols = [q0, q1, m01, m23, m45, m67] + g
    for s, v in enumerate(cols):
        out_ref[0, :, s * C:(s + 1) * C] = v


@jax.jit
def kernel(x, W1, b1, W2, b2, W3, b3):
    B, t, n, c = x.shape
    BT = B * t
    xyzS = x.reshape(BT, n, c)
    xyzT = jnp.transpose(xyzS, (0, 2, 1))
    b1r = b1.reshape(1, OUT_CH)
    b2r = b2.reshape(1, OUT_CH)
    b3r = b3.reshape(1, OUT_CH)

    out = pl.pallas_call(
        _tc_kernel,
        grid=(BT, n // CT),
        in_specs=[
            pl.BlockSpec((1, 3, n), lambda b, ct: (b, 0, 0)),
            pl.BlockSpec((1, n, 3), lambda b, ct: (b, 0, 0)),
            pl.BlockSpec((OUT_CH, 3), lambda b, ct: (0, 0)),
            pl.BlockSpec((OUT_CH, OUT_CH), lambda b, ct: (0, 0)),
            pl.BlockSpec((OUT_CH, OUT_CH), lambda b, ct: (0, 0)),
            pl.BlockSpec((1, OUT_CH), lambda b, ct: (0, 0)),
            pl.BlockSpec((1, OUT_CH), lambda b, ct: (0, 0)),
            pl.BlockSpec((1, OUT_CH), lambda b, ct: (0, 0)),
        ],
        out_specs=pl.BlockSpec((1, CT, 14 * OUT_CH), lambda b, ct: (b, ct, 0)),
        out_shape=jax.ShapeDtypeStruct((BT, n, 14 * OUT_CH), jnp.float32),
        compiler_params=pltpu.CompilerParams(
            dimension_semantics=("parallel", "parallel")),
    )(xyzT, xyzS, W1, W2, W3, b1r, b2r, b3r)
    return out.reshape(B, t, n, 14 * OUT_CH)


# speedup vs baseline: 1.4122x; 1.0698x over previous
import jax
import jax.numpy as jnp
from jax.experimental import pallas as pl
from jax.experimental.pallas import tpu as pltpu

RADIUS = 0.2
N = 1024
CT = 1024
OUT_CH = 128


def _tc_kernel(xyzT_ref, xyzS_ref, W1_ref, W2_ref, W3_ref,
               b1_ref, b2_ref, b3_ref, out_ref):
    ct = pl.program_id(1)
    judge = jnp.float32(RADIUS * RADIUS)

    W21 = jnp.dot(W2_ref[...], W1_ref[...], preferred_element_type=jnp.float32, precision=jax.lax.Precision.HIGHEST)
    W321 = jnp.dot(W3_ref[...], W21, preferred_element_type=jnp.float32, precision=jax.lax.Precision.HIGHEST)
    be = jax.lax.dot_general(b1_ref[...], W2_ref[...],
                             (((1,), (1,)), ((), ())), precision=jax.lax.Precision.HIGHEST) + b2_ref[...]
    beff = jax.lax.dot_general(be, W3_ref[...],
                               (((1,), (1,)), ((), ())), precision=jax.lax.Precision.HIGHEST) + b3_ref[...]

    xyz_all = xyzS_ref[0]
    P = jax.lax.dot_general(xyz_all, W321, (((1,), (1,)), ((), ())),
                            preferred_element_type=jnp.float32, precision=jax.lax.Precision.HIGHEST)
    P_hi = P.astype(jnp.bfloat16)
    P_lo = (P - P_hi.astype(jnp.float32)).astype(jnp.bfloat16)

    xs = xyzT_ref[0, 0:1, :]
    ys = xyzT_ref[0, 1:2, :]
    zs = xyzT_ref[0, 2:3, :]
    c3 = xyzS_ref[0, pl.ds(ct * CT, CT), :]
    xc = c3[:, 0:1]
    yc = c3[:, 1:2]
    zc = c3[:, 2:3]

    dx = xs - xc
    dy = ys - yc
    dz = zs - zc
    dist = dx * dx + dy * dy + dz * dz

    one = jnp.float32(1.0)
    i0 = jnp.int32(0)
    neg_eps = jnp.float32(-(2.0 ** -25))
    s = (jnp.where(dx >= neg_eps, jnp.int32(4), i0)
         + jnp.where(dy >= neg_eps, jnp.int32(2), i0)
         + jnp.where(dz >= neg_eps, jnp.int32(1), i0))
    valid = (dist > jnp.float32(1e-10)) & (dist < judge)

    lane = jax.lax.broadcasted_iota(jnp.int32, (CT, N), 1)
    row = jax.lax.broadcasted_iota(jnp.int32, (CT, N), 0)
    eye = lane == (row + ct * CT)
    base = jnp.where(eye, judge, jnp.float32(1e10))
    dv = jnp.where(valid, dist, base)

    Pc = jax.lax.dot_general(c3, W321, (((1,), (1,)), ((), ())),
                             preferred_element_type=jnp.float32, precision=jax.lax.Precision.HIGHEST)
    offs = beff - Pc

    g = []
    for i in range(8):
        d_i = jnp.where(s == i, dv, base)
        idx = jnp.argmin(d_i, axis=1)[:, None].astype(jnp.int32)
        onehot = (lane == idx).astype(jnp.bfloat16)
        Gi = (jnp.dot(onehot, P_hi, preferred_element_type=jnp.float32)
              + jnp.dot(onehot, P_lo, preferred_element_type=jnp.float32))
        g.append(Gi + offs)

    m01 = jnp.maximum(g[0], g[1])
    m23 = jnp.maximum(g[2], g[3])
    m45 = jnp.maximum(g[4], g[5])
    m67 = jnp.maximum(g[6], g[7])
    q0 = jnp.maximum(m01, m23)
    q1 = jnp.maximum(m45, m67)

    C = OUT_CH
    cols = [q0, q1, m01, m23, m45, m67] + g
    for s, v in enumerate(cols):
        out_ref[0, :, s * C:(s + 1) * C] = v


@jax.jit
def kernel(x, W1, b1, W2, b2, W3, b3):
    B, t, n, c = x.shape
    BT = B * t
    xyzS = x.reshape(BT, n, c)
    xyzT = jnp.transpose(xyzS, (0, 2, 1))
    b1r = b1.reshape(1, OUT_CH)
    b2r = b2.reshape(1, OUT_CH)
    b3r = b3.reshape(1, OUT_CH)

    out = pl.pallas_call(
        _tc_kernel,
        grid=(BT, n // CT),
        in_specs=[
            pl.BlockSpec((1, 3, n), lambda b, ct: (b, 0, 0)),
            pl.BlockSpec((1, n, 3), lambda b, ct: (b, 0, 0)),
            pl.BlockSpec((OUT_CH, 3), lambda b, ct: (0, 0)),
            pl.BlockSpec((OUT_CH, OUT_CH), lambda b, ct: (0, 0)),
            pl.BlockSpec((OUT_CH, OUT_CH), lambda b, ct: (0, 0)),
            pl.BlockSpec((1, OUT_CH), lambda b, ct: (0, 0)),
            pl.BlockSpec((1, OUT_CH), lambda b, ct: (0, 0)),
            pl.BlockSpec((1, OUT_CH), lambda b, ct: (0, 0)),
        ],
        out_specs=pl.BlockSpec((1, CT, 14 * OUT_CH), lambda b, ct: (b, ct, 0)),
        out_shape=jax.ShapeDtypeStruct((BT, n, 14 * OUT_CH), jnp.float32),
        compiler_params=pltpu.CompilerParams(
            dimension_semantics=("parallel", "parallel")),
    )(xyzT, xyzS, W1, W2, W3, b1r, b2r, b3r)
    return out.reshape(B, t, n, 14 * OUT_CH)
